# baseline (speedup 1.0000x reference)
import functools

import jax
import jax.numpy as jnp
from jax import lax
from jax.experimental import pallas as pl
from jax.experimental.pallas import tpu as pltpu
from jax.experimental.pallas import tpu_sc as plsc

_D = 32
_BATCH = 16384
_HIST = 200
_B = _BATCH * _HIST
_NPART = 8
_PB = _BATCH // _NPART
_QB = _PB * _HIST
_NC = 2
_NS = 16
_NW = _NC * _NS
_BPW = _QB // _NW
_CH = 800
_NBUF = 4
_NCHUNK = _BPW // _CH

_mesh = plsc.VectorSubcoreMesh(core_axis_name="c", subcore_axis_name="s")


def _make_gather(part):
  @functools.partial(
      pl.kernel,
      out_type=jax.ShapeDtypeStruct((_QB, _D), jnp.float32),
      mesh=_mesh,
      scratch_types=[
          pltpu.VMEM((_NBUF, _CH), jnp.int32),
          pltpu.VMEM((_NBUF, _CH, _D), jnp.float32),
          pltpu.SemaphoreType.DMA((_NBUF,)),
          pltpu.SemaphoreType.DMA((_NBUF,)),
          pltpu.SemaphoreType.DMA((_NBUF,)),
      ],
      compiler_params=pltpu.CompilerParams(use_tc_tiling_on_sc=False),
  )
  def _gather_kernel(idx_hbm, tab_hbm, out_hbm, idx_v, rows_v, sem_i, sem_g,
                     sem_o):
    wid = lax.axis_index("s") * _NC + lax.axis_index("c")
    obase = wid * _BPW
    ibase = part * _QB + wid * _BPW

    def start_idx(g, b):
      pltpu.async_copy(
          idx_hbm.at[pl.ds(ibase + g * _CH, _CH)], idx_v.at[b], sem_i.at[b])

    def wait_idx(b):
      pltpu.make_async_copy(
          idx_hbm.at[pl.ds(0, _CH)], idx_v.at[b], sem_i.at[b]).wait()

    def start_gather(b):
      pltpu.async_copy(tab_hbm.at[idx_v.at[b]], rows_v.at[b], sem_g.at[b])

    def wait_gather(b):
      pltpu.make_async_copy(
          tab_hbm.at[idx_v.at[b]], rows_v.at[b], sem_g.at[b]).wait()

    def start_out(g, b):
      pltpu.async_copy(
          rows_v.at[b], out_hbm.at[pl.ds(obase + g * _CH, _CH)], sem_o.at[b])

    def wait_out(b):
      pltpu.make_async_copy(
          rows_v.at[b], out_hbm.at[pl.ds(0, _CH)], sem_o.at[b]).wait()

    for b in range(_NBUF):
      start_idx(b, b)
    wait_idx(0)
    start_gather(0)

    @pl.loop(0, _NCHUNK, step=_NBUF)
    def _outer(g0):
      for b in range(_NBUF):
        g = g0 + b
        bn = (b + 1) % _NBUF

        @pl.when(g + 1 < _NCHUNK)
        def _():
          wait_idx(bn)

          @pl.when(g + 1 >= _NBUF)
          def _():
            wait_out(bn)

          start_gather(bn)

        wait_gather(b)
        start_out(g, b)

        @pl.when(g + _NBUF < _NCHUNK)
        def _():
          start_idx(g + _NBUF, b)

    for b in range(_NBUF):
      wait_out(b)

  return _gather_kernel


_HG = 4
_NHG = _HIST // _HG


def _transpose_body(in_ref, out_ref):
  t = jnp.swapaxes(in_ref[:, 0, 0, :], 0, 1)
  out_ref[...] = t.reshape(_HG, _D, _PB)


def _transpose_body_acc(in_ref, prev_ref, out_ref):
  del prev_ref
  _transpose_body(in_ref, out_ref)


def _transpose_part(part, rm4, prev):
  in_specs = [pl.BlockSpec((_PB, 1, 1, 128), lambda j: (0, j, 0, 0))]
  args = (rm4,)
  body = _transpose_body
  aliases = {}
  if prev is not None:
    in_specs.append(pl.BlockSpec(memory_space=pl.ANY))
    args = (rm4, prev)
    body = _transpose_body_acc
    aliases = {1: 0}
  return pl.pallas_call(
      body,
      grid=(_NHG,),
      in_specs=in_specs,
      out_specs=pl.BlockSpec((_HG, _D, _PB), lambda j: (j, 0, part)),
      out_shape=jax.ShapeDtypeStruct((_HIST, _D, _BATCH), jnp.float32),
      input_output_aliases=aliases,
  )(*args)


def kernel(idx, weight):
  flat = idx.reshape(-1).astype(jnp.int32)
  out = None
  for p in range(_NPART):
    rm = _make_gather(p)(flat, weight)
    rm4 = rm.reshape(_PB, _NHG, 1, 128)
    out = _transpose_part(p, rm4, out)
  return out.transpose(2, 0, 1)
